# baseline (speedup 1.0000x reference)
import functools

import jax
import jax.numpy as jnp
from jax import lax
from jax.experimental import pallas as pl
from jax.experimental.pallas import tpu as pltpu
from jax.experimental.pallas import tpu_sc as plsc

N_ANCHOR = 20000
N_GT = 64
POS_IOU = 0.7
NEG_IOU = 0.3
BIG_I = 2**30

L = 16
N_CHUNKS = 16
CHUNK = 1264
N_VEC = CHUNK // L
GTB = 4
N_GTB = N_GT // GTB
N_PAD_SC = N_CHUNKS * CHUNK

N_PAD_TC = 20480

_f32 = jnp.float32
_i32 = jnp.int32
_LN2 = 0.6931471805599453
_SQRT2 = 1.4142135623730951


def _vlog(x):
    bits = plsc.bitcast(x, _i32)
    e = jnp.right_shift(bits, 23) - 127
    m = plsc.bitcast(jnp.bitwise_or(jnp.bitwise_and(bits, 0x7FFFFF),
                                    0x3F800000), _f32)
    big = m > _SQRT2
    m = jnp.where(big, m * 0.5, m)
    e = jnp.where(big, e + 1, e)
    z = (m - 1.0) / (m + 1.0)
    z2 = z * z
    p = ((z2 * (1.0 / 7.0) + (1.0 / 5.0)) * z2 + (1.0 / 3.0)) * z2 + 1.0
    return e.astype(_f32) * _LN2 + 2.0 * z * p


def _sl1(d):
    return jnp.where(d < 1.0, 0.5 * d * d, d - 0.5)


def _sc_body(anc, bbox, loc, score, out,
             ax1, ay1, ax2, ay2, l0, l1, l2, l3, sc0, sc1,
             bx1, by1, bx2, by2, areaa, aidxs,
             aos_a, aos_l, aos_s, aos_b,
             miou, rga, flags, colv, coli, scat,
             mv, mi, ms, gt_arg, stage, fin, outv, sem,
             sh_colv, sh_coli, sh_scat, sh_sums):
    c = lax.axis_index("c")
    s = lax.axis_index("s")
    img = c
    base = s * CHUNK
    lanes = lax.broadcasted_iota(_i32, (L,), 0)

    NP = N_PAD_SC
    copies = [
        pltpu.make_async_copy(anc.at[pl.ds(base * 4, CHUNK * 4)], aos_a, sem),
        pltpu.make_async_copy(loc.at[pl.ds((img * NP + base) * 4, CHUNK * 4)],
                              aos_l, sem),
        pltpu.make_async_copy(score.at[pl.ds((img * NP + base) * 2, CHUNK * 2)],
                              aos_s, sem),
        pltpu.make_async_copy(bbox.at[pl.ds(img * N_GT * 4, N_GT * 4)],
                              aos_b, sem),
    ]
    for cp in copies:
        cp.start()

    def _init_scat(j, _):
        scat[pl.ds(j * L, L)] = jnp.full((L,), -1, _i32)
        return 0

    lax.fori_loop(0, N_GT, _init_scat, 0)

    def _init_flags(i, _):
        flags[pl.ds(i * L, L)] = jnp.zeros((L,), _i32)
        return 0

    lax.fori_loop(0, N_VEC, _init_flags, 0)
    for cp in copies:
        cp.wait()

    l4 = lanes * 4
    l2_ = lanes * 2

    def _init_pre(i, _):
        off = i * L
        a1 = plsc.load_gather(aos_a, [off * 4 + l4 + 0])
        a2 = plsc.load_gather(aos_a, [off * 4 + l4 + 1])
        a3 = plsc.load_gather(aos_a, [off * 4 + l4 + 2])
        a4 = plsc.load_gather(aos_a, [off * 4 + l4 + 3])
        ax1[pl.ds(off, L)] = a1
        ay1[pl.ds(off, L)] = a2
        ax2[pl.ds(off, L)] = a3
        ay2[pl.ds(off, L)] = a4
        l0[pl.ds(off, L)] = plsc.load_gather(aos_l, [off * 4 + l4 + 0])
        l1[pl.ds(off, L)] = plsc.load_gather(aos_l, [off * 4 + l4 + 1])
        l2[pl.ds(off, L)] = plsc.load_gather(aos_l, [off * 4 + l4 + 2])
        l3[pl.ds(off, L)] = plsc.load_gather(aos_l, [off * 4 + l4 + 3])
        sc0[pl.ds(off, L)] = plsc.load_gather(aos_s, [off * 2 + l2_ + 0])
        sc1[pl.ds(off, L)] = plsc.load_gather(aos_s, [off * 2 + l2_ + 1])
        areaa[pl.ds(off, L)] = (a3 - a1) * (a4 - a2)
        aidxs[pl.ds(off, L)] = base + off + lanes
        return 0

    lax.fori_loop(0, N_VEC, _init_pre, 0)

    for k in range(N_GT // L):
        off = k * L
        bx1[pl.ds(off, L)] = plsc.load_gather(aos_b, [off * 4 + l4 + 0])
        by1[pl.ds(off, L)] = plsc.load_gather(aos_b, [off * 4 + l4 + 1])
        bx2[pl.ds(off, L)] = plsc.load_gather(aos_b, [off * 4 + l4 + 2])
        by2[pl.ds(off, L)] = plsc.load_gather(aos_b, [off * 4 + l4 + 3])

    for gtb in range(N_GTB):
        blk = (gtb * GTB) // L
        off16 = blk * L
        sub = (gtb * GTB) % L
        b1v = bx1[pl.ds(off16, L)]
        b2v = by1[pl.ds(off16, L)]
        b3v = bx2[pl.ds(off16, L)]
        b4v = by2[pl.ds(off16, L)]
        abv = (b3v - b1v) * (b4v - b2v)
        zsplat = jnp.zeros((L,), _f32)
        gb1 = [zsplat + b1v[sub + j] for j in range(GTB)]
        gb2 = [zsplat + b2v[sub + j] for j in range(GTB)]
        gb3 = [zsplat + b3v[sub + j] for j in range(GTB)]
        gb4 = [zsplat + b4v[sub + j] for j in range(GTB)]
        gab = [zsplat + abv[sub + j] for j in range(GTB)]

        def _main(i, col, gtb=gtb, gb1=gb1, gb2=gb2, gb3=gb3, gb4=gb4,
                  gab=gab):
            off = i * L
            a1 = ax1[pl.ds(off, L)]
            a2 = ay1[pl.ds(off, L)]
            a3 = ax2[pl.ds(off, L)]
            a4 = ay2[pl.ds(off, L)]
            aidx = aidxs[pl.ds(off, L)]
            area_a = areaa[pl.ds(off, L)]
            if gtb == 0:
                rmax = jnp.full((L,), -1.0, _f32)
                rg = jnp.zeros((L,), _i32)
            else:
                rmax = miou[pl.ds(off, L)]
                rg = rga[pl.ds(off, L)]
            cvs = list(col)
            for j in range(GTB):
                g = gtb * GTB + j
                iw = jnp.maximum(
                    jnp.minimum(a3, gb3[j]) - jnp.maximum(a1, gb1[j]), 0.0)
                ih = jnp.maximum(
                    jnp.minimum(a4, gb4[j]) - jnp.maximum(a2, gb2[j]), 0.0)
                inter = iw * ih
                iou = inter / (area_a + gab[j] - inter + 1e-9)
                better = iou > rmax
                rmax = jnp.where(better, iou, rmax)
                rg = jnp.where(better, g, rg)
                cb = iou > cvs[2 * j]
                cvs[2 * j] = jnp.where(cb, iou, cvs[2 * j])
                cvs[2 * j + 1] = jnp.where(cb, aidx, cvs[2 * j + 1])
            miou[pl.ds(off, L)] = rmax
            rga[pl.ds(off, L)] = rg
            return tuple(cvs)

        col0 = []
        for j in range(GTB):
            col0.append(jnp.full((L,), -1.0, _f32))
            col0.append(jnp.zeros((L,), _i32))
        colf = lax.fori_loop(0, N_VEC, _main, tuple(col0))
        for j in range(GTB):
            g = gtb * GTB + j
            colv[pl.ds(g * L, L)] = colf[2 * j]
            coli[pl.ds(g * L, L)] = colf[2 * j + 1]

    def _scatp(i, _):
        off = i * L
        rg = rga[pl.ds(off, L)]
        valid = aidxs[pl.ds(off, L)] < N_ANCHOR
        plsc.store_scatter(scat, [rg * L + lanes],
                           aidxs[pl.ds(off, L)], mask=valid)
        return 0

    lax.fori_loop(0, N_VEC, _scatp, 0)

    TBL = N_GT * L
    pltpu.sync_copy(colv, sh_colv.at[pl.ds(s * TBL, TBL)])
    pltpu.sync_copy(coli, sh_coli.at[pl.ds(s * TBL, TBL)])
    pltpu.sync_copy(scat, sh_scat.at[pl.ds(s * TBL, TBL)])
    plsc.subcore_barrier()

    pltpu.sync_copy(sh_colv, mv)
    pltpu.sync_copy(sh_coli, mi)
    pltpu.sync_copy(sh_scat, ms)

    lane0 = lanes == 0
    zi = jnp.zeros((L,), _i32)

    def _merge(g, _):
        off = g * L
        bv = mv[pl.ds(off, L)]
        bi = mi[pl.ds(off, L)]
        sm = ms[pl.ds(off, L)]
        for ch in range(1, N_CHUNKS):
            coff = ch * TBL + off
            cv = mv[pl.ds(coff, L)]
            ci = mi[pl.ds(coff, L)]
            cb = cv > bv
            bv = jnp.where(cb, cv, bv)
            bi = jnp.where(cb, ci, bi)
            sm = jnp.maximum(sm, ms[pl.ds(coff, L)])
        cmax = jnp.max(bv)
        cidx = jnp.min(jnp.where(bv == cmax, bi, BIG_I))
        sg = jnp.max(sm)
        ga = jnp.where(sg >= 0, sg, cidx)
        plsc.store_scatter(gt_arg, [zi + g], zi + ga, mask=lane0)
        return 0

    lax.fori_loop(0, N_GT, _merge, 0)

    ones_i = jnp.ones((L,), _i32)
    for gb in range(N_GT // L):
        ga_v = gt_arg[pl.ds(gb * L, L)]
        inm = (ga_v >= base) & (ga_v < base + CHUNK)
        li = jnp.where(inm, ga_v - base, 0)
        plsc.store_scatter(flags, [li], ones_i, mask=inm)

    def _delta(i, acc):
        a_pos, a_rl, a_val, a_ce = acc
        off = i * L
        valid = aidxs[pl.ds(off, L)] < N_ANCHOR
        fl = flags[pl.ds(off, L)] > 0
        mi_v = miou[pl.ds(off, L)]
        posm = (mi_v >= POS_IOU) | fl
        validm = posm | ((mi_v < NEG_IOU) & valid)
        rg = rga[pl.ds(off, L)]
        a1 = ax1[pl.ds(off, L)]
        a2 = ay1[pl.ds(off, L)]
        a3 = ax2[pl.ds(off, L)]
        a4 = ay2[pl.ds(off, L)]
        m1 = plsc.load_gather(bx1, [rg])
        m2 = plsc.load_gather(by1, [rg])
        m3 = plsc.load_gather(bx2, [rg])
        m4 = plsc.load_gather(by2, [rg])
        eps = jnp.finfo(_f32).eps
        w = a3 - a1
        h = a4 - a2
        cx = a1 + w * 0.5
        cy = a2 + h * 0.5
        dw_ = m3 - m1
        dh_ = m4 - m2
        dcx = m1 + dw_ * 0.5
        dcy = m2 + dh_ * 0.5
        w = jnp.maximum(w, eps)
        h = jnp.maximum(h, eps)
        tdx = (dcx - cx) / w
        tdy = (dcy - cy) / h
        tdw = _vlog(dw_ / w)
        tdh = _vlog(dh_ / h)
        rl = (_sl1(jnp.abs(tdx - l0[pl.ds(off, L)]))
              + _sl1(jnp.abs(tdy - l1[pl.ds(off, L)]))
              + _sl1(jnp.abs(tdw - l2[pl.ds(off, L)]))
              + _sl1(jnp.abs(tdh - l3[pl.ds(off, L)])))
        s0 = sc0[pl.ds(off, L)]
        s1 = sc1[pl.ds(off, L)]
        mx = jnp.maximum(s0, s1)
        lse = mx + _vlog(1.0 + jnp.exp(-jnp.abs(s0 - s1)))
        a_pos = a_pos + jnp.where(posm, 1.0, 0.0)
        a_rl = a_rl + jnp.where(posm, rl, 0.0)
        a_val = a_val + jnp.where(validm, 1.0, 0.0)
        ce = lse - jnp.where(posm, s1, s0)
        a_ce = a_ce + jnp.where(validm, ce, 0.0)
        return (a_pos, a_rl, a_val, a_ce)

    zero = jnp.zeros((L,), _f32)
    a_pos, a_rl, a_val, a_ce = lax.fori_loop(
        0, N_VEC, _delta, (zero, zero, zero, zero))
    stage[pl.ds(0, L)] = a_pos
    stage[pl.ds(L, L)] = a_rl
    stage[pl.ds(2 * L, L)] = a_val
    stage[pl.ds(3 * L, L)] = a_ce
    pltpu.sync_copy(stage, sh_sums.at[pl.ds(s * 4 * L, 4 * L)])
    plsc.subcore_barrier()

    @pl.when(s == 0)
    def _finalize():
        pltpu.sync_copy(sh_sums, fin)
        t_pos = jnp.zeros((L,), _f32)
        t_rl = jnp.zeros((L,), _f32)
        t_val = jnp.zeros((L,), _f32)
        t_ce = jnp.zeros((L,), _f32)
        for ch in range(N_CHUNKS):
            o = ch * 4 * L
            t_pos = t_pos + fin[pl.ds(o, L)]
            t_rl = t_rl + fin[pl.ds(o + L, L)]
            t_val = t_val + fin[pl.ds(o + 2 * L, L)]
            t_ce = t_ce + fin[pl.ds(o + 3 * L, L)]
        zf = jnp.zeros((L,), _f32)
        num_pos = jnp.maximum(zf + jnp.sum(t_pos), 1.0)
        num_val = jnp.maximum(zf + jnp.sum(t_val), 1.0)
        total = ((zf + jnp.sum(t_rl)) / num_pos
                 + (zf + jnp.sum(t_ce)) / num_val)
        outv[...] = total
        pltpu.sync_copy(outv, out.at[pl.ds(c * L, L)])


def _sc_call(anc, bbox, loc, score):
    mesh = plsc.VectorSubcoreMesh(core_axis_name="c", subcore_axis_name="s",
                                  num_cores=2, num_subcores=16)
    return pl.kernel(
        _sc_body,
        out_type=jax.ShapeDtypeStruct((2 * L,), _f32),
        mesh=mesh,
        compiler_params=pltpu.CompilerParams(needs_layout_passes=False),
        scratch_types=[
            pltpu.VMEM((CHUNK,), _f32), pltpu.VMEM((CHUNK,), _f32),
            pltpu.VMEM((CHUNK,), _f32), pltpu.VMEM((CHUNK,), _f32),
            pltpu.VMEM((CHUNK,), _f32), pltpu.VMEM((CHUNK,), _f32),
            pltpu.VMEM((CHUNK,), _f32), pltpu.VMEM((CHUNK,), _f32),
            pltpu.VMEM((CHUNK,), _f32), pltpu.VMEM((CHUNK,), _f32),
            pltpu.VMEM((N_GT,), _f32), pltpu.VMEM((N_GT,), _f32),
            pltpu.VMEM((N_GT,), _f32), pltpu.VMEM((N_GT,), _f32),
            pltpu.VMEM((CHUNK,), _f32), pltpu.VMEM((CHUNK,), _i32),
            pltpu.VMEM((CHUNK * 4,), _f32), pltpu.VMEM((CHUNK * 4,), _f32),
            pltpu.VMEM((CHUNK * 2,), _f32), pltpu.VMEM((N_GT * 4,), _f32),
            pltpu.VMEM((CHUNK,), _f32), pltpu.VMEM((CHUNK,), _i32),
            pltpu.VMEM((CHUNK,), _i32),
            pltpu.VMEM((N_GT * L,), _f32), pltpu.VMEM((N_GT * L,), _i32),
            pltpu.VMEM((N_GT * L,), _i32),
            pltpu.VMEM((N_CHUNKS * N_GT * L,), _f32),
            pltpu.VMEM((N_CHUNKS * N_GT * L,), _i32),
            pltpu.VMEM((N_CHUNKS * N_GT * L,), _i32),
            pltpu.VMEM((N_GT,), _i32),
            pltpu.VMEM((4 * L,), _f32),
            pltpu.VMEM((N_CHUNKS * 4 * L,), _f32),
            pltpu.VMEM((L,), _f32),
            pltpu.SemaphoreType.DMA,
            pltpu.VMEM_SHARED((N_CHUNKS * N_GT * L,), _f32),
            pltpu.VMEM_SHARED((N_CHUNKS * N_GT * L,), _i32),
            pltpu.VMEM_SHARED((N_CHUNKS * N_GT * L,), _i32),
            pltpu.VMEM_SHARED((N_CHUNKS * 4 * L,), _f32),
        ],
    )(anc, bbox, loc, score)


def _tc_body(anchors_ref, bbox_ref, loc_ref, score_ref, out_ref):
    NP = N_PAD_TC
    ax1 = anchors_ref[0:1, :]
    ay1 = anchors_ref[1:2, :]
    ax2 = anchors_ref[2:3, :]
    ay2 = anchors_ref[3:4, :]
    bt = bbox_ref[0]
    bx1 = bt[0:1, :].reshape(N_GT, 1)
    by1 = bt[1:2, :].reshape(N_GT, 1)
    bx2 = bt[2:3, :].reshape(N_GT, 1)
    by2 = bt[3:4, :].reshape(N_GT, 1)

    tlx = jnp.maximum(ax1, bx1)
    tly = jnp.maximum(ay1, by1)
    brx = jnp.minimum(ax2, bx2)
    bry = jnp.minimum(ay2, by2)
    iw = jnp.maximum(brx - tlx, 0.0)
    ih = jnp.maximum(bry - tly, 0.0)
    inter = iw * ih
    area_a = (ax2 - ax1) * (ay2 - ay1)
    area_b = (bx2 - bx1) * (by2 - by1)
    iou = inter / (area_a + area_b - inter + 1e-9)

    i_iota = lax.broadcasted_iota(jnp.int32, (N_GT, NP), 1)
    g_iota = lax.broadcasted_iota(jnp.int32, (N_GT, NP), 0)

    max_iou = jnp.max(iou, axis=0, keepdims=True)
    argmax_g = jnp.min(jnp.where(iou == max_iou, g_iota, N_GT),
                       axis=0, keepdims=True)

    colmax = jnp.max(iou, axis=1, keepdims=True)
    col_argmax = jnp.min(jnp.where(iou == colmax, i_iota, NP),
                         axis=1, keepdims=True)

    lane_valid = i_iota < N_ANCHOR
    scat = jnp.max(jnp.where((argmax_g == g_iota) & lane_valid, i_iota, -1),
                   axis=1, keepdims=True)
    gt_argmax = jnp.where(scat >= 0, scat, col_argmax)

    valid_lane = lax.broadcasted_iota(jnp.int32, (1, NP), 1) < N_ANCHOR
    member = jnp.max(jnp.where(gt_argmax == i_iota, 1, 0),
                     axis=0, keepdims=True) > 0
    pos = (max_iou >= POS_IOU) | member
    neg = (max_iou < NEG_IOU) & valid_lane
    valid = pos | neg

    onehot = argmax_g == g_iota
    mx1 = jnp.max(jnp.where(onehot, bx1, -1e30), axis=0, keepdims=True)
    my1 = jnp.max(jnp.where(onehot, by1, -1e30), axis=0, keepdims=True)
    mx2 = jnp.max(jnp.where(onehot, bx2, -1e30), axis=0, keepdims=True)
    my2 = jnp.max(jnp.where(onehot, by2, -1e30), axis=0, keepdims=True)

    eps = jnp.finfo(jnp.float32).eps
    w = ax2 - ax1
    h = ay2 - ay1
    cx = ax1 + w * 0.5
    cy = ay1 + h * 0.5
    dw_ = mx2 - mx1
    dh_ = my2 - my1
    dcx = mx1 + dw_ * 0.5
    dcy = my1 + dh_ * 0.5
    w = jnp.maximum(w, eps)
    h = jnp.maximum(h, eps)
    tdx = (dcx - cx) / w
    tdy = (dcy - cy) / h
    tdw = jnp.log(dw_ / w)
    tdh = jnp.log(dh_ / h)

    lr = loc_ref[0]
    d0 = jnp.abs(tdx - lr[0:1, :])
    d1 = jnp.abs(tdy - lr[1:2, :])
    d2 = jnp.abs(tdw - lr[2:3, :])
    d3 = jnp.abs(tdh - lr[3:4, :])
    rl = _sl1(d0) + _sl1(d1) + _sl1(d2) + _sl1(d3)
    posf = pos.astype(jnp.float32)
    num_pos = jnp.maximum(jnp.sum(posf), 1.0)
    loc_loss = jnp.sum(rl * posf) / num_pos

    s0 = score_ref[0][0:1, :]
    s1 = score_ref[0][1:2, :]
    m = jnp.maximum(s0, s1)
    lse = m + jnp.log(jnp.exp(s0 - m) + jnp.exp(s1 - m))
    ce = lse - jnp.where(pos, s1, s0)
    validf = valid.astype(jnp.float32)
    num_valid = jnp.maximum(jnp.sum(validf), 1.0)
    cls_loss = jnp.sum(jnp.where(valid, ce, 0.0)) / num_valid

    out_ref[:, :, :] = (loc_loss + cls_loss).reshape(1, 1, 1)


def _tc_call(anchors_t, bboxes_t, loc_t, score_t, n_img):
    return pl.pallas_call(
        _tc_body,
        grid=(n_img,),
        in_specs=[
            pl.BlockSpec((4, N_PAD_TC), lambda b: (0, 0)),
            pl.BlockSpec((1, 4, N_GT), lambda b: (b, 0, 0)),
            pl.BlockSpec((1, 4, N_PAD_TC), lambda b: (b, 0, 0)),
            pl.BlockSpec((1, 2, N_PAD_TC), lambda b: (b, 0, 0)),
        ],
        out_specs=pl.BlockSpec((1, 1, 1), lambda b: (b, 0, 0)),
        out_shape=jax.ShapeDtypeStruct((n_img, 1, 1), _f32),
    )(anchors_t, bboxes_t, loc_t, score_t)


@jax.jit
def kernel(anchors, bboxes, rpn_loc, rpn_score):
    anchors = anchors.astype(_f32)
    bboxes = bboxes.astype(_f32)
    pad_sc = N_PAD_SC - N_ANCHOR
    anc_sc = jnp.pad(anchors.reshape(-1), (0, pad_sc * 4))
    bbox_sc = bboxes[2:].reshape(-1)
    loc_sc = jnp.pad(rpn_loc[2:].reshape(2, -1),
                     ((0, 0), (0, pad_sc * 4))).reshape(-1)
    score_sc = jnp.pad(rpn_score[2:].reshape(2, -1),
                       ((0, 0), (0, pad_sc * 2))).reshape(-1)
    sc_out = _sc_call(anc_sc, bbox_sc, loc_sc, score_sc)

    pad_tc = N_PAD_TC - N_ANCHOR
    anc_tc = jnp.pad(anchors.T, ((0, 0), (0, pad_tc)))
    bbox_tc = jnp.transpose(bboxes[:2], (0, 2, 1))
    loc_tc = jnp.pad(jnp.transpose(rpn_loc[:2], (0, 2, 1)),
                     ((0, 0), (0, 0), (0, pad_tc)))
    score_tc = jnp.pad(jnp.transpose(rpn_score[:2], (0, 2, 1)),
                       ((0, 0), (0, 0), (0, pad_tc)))
    tc_out = _tc_call(anc_tc, bbox_tc, loc_tc, score_tc, 2)

    return tc_out[0, 0, 0] + tc_out[1, 0, 0] + sc_out[0] + sc_out[L]


# speedup vs baseline: 2.5558x; 2.5558x over previous
import functools

import jax
import jax.numpy as jnp
from jax import lax
from jax.experimental import pallas as pl
from jax.experimental.pallas import tpu as pltpu
from jax.experimental.pallas import tpu_sc as plsc

N_ANCHOR = 20000
N_GT = 64
POS_IOU = 0.7
NEG_IOU = 0.3
BIG_I = 2**30

L = 16
N_CHUNKS = 16
CHUNK = 1264
N_VEC = CHUNK // L
GTB = 4
N_GTB = N_GT // GTB
N_PAD_SC = N_CHUNKS * CHUNK

N_PAD_TC = 20480

_f32 = jnp.float32
_i32 = jnp.int32
_LN2 = 0.6931471805599453
_SQRT2 = 1.4142135623730951


def _vlog(x):
    bits = plsc.bitcast(x, _i32)
    e = jnp.right_shift(bits, 23) - 127
    m = plsc.bitcast(jnp.bitwise_or(jnp.bitwise_and(bits, 0x7FFFFF),
                                    0x3F800000), _f32)
    big = m > _SQRT2
    m = jnp.where(big, m * 0.5, m)
    e = jnp.where(big, e + 1, e)
    z = (m - 1.0) / (m + 1.0)
    z2 = z * z
    p = ((z2 * (1.0 / 7.0) + (1.0 / 5.0)) * z2 + (1.0 / 3.0)) * z2 + 1.0
    return e.astype(_f32) * _LN2 + 2.0 * z * p


def _sl1(d):
    return jnp.where(d < 1.0, 0.5 * d * d, d - 0.5)


def _sc_body(anc, bbox, loc, score, out,
             ax1, ay1, ax2, ay2, l0, l1, l2, l3, sc0, sc1,
             bx1, by1, bx2, by2, areaa, aidxs,
             miou, rga, flags, colv, coli, scat,
             mv, mi, ms, gt_arg, stage, fin, outv, sem,
             sh_colv, sh_coli, sh_scat, sh_sums):
    c = lax.axis_index("c")
    s = lax.axis_index("s")
    img = c
    base = s * CHUNK
    lanes = lax.broadcasted_iota(_i32, (L,), 0)

    NP = N_PAD_SC
    copies = [
        pltpu.make_async_copy(anc.at[pl.ds(0 * NP + base, CHUNK)], ax1, sem),
        pltpu.make_async_copy(anc.at[pl.ds(1 * NP + base, CHUNK)], ay1, sem),
        pltpu.make_async_copy(anc.at[pl.ds(2 * NP + base, CHUNK)], ax2, sem),
        pltpu.make_async_copy(anc.at[pl.ds(3 * NP + base, CHUNK)], ay2, sem),
        pltpu.make_async_copy(loc.at[pl.ds((img * 4 + 0) * NP + base, CHUNK)], l0, sem),
        pltpu.make_async_copy(loc.at[pl.ds((img * 4 + 1) * NP + base, CHUNK)], l1, sem),
        pltpu.make_async_copy(loc.at[pl.ds((img * 4 + 2) * NP + base, CHUNK)], l2, sem),
        pltpu.make_async_copy(loc.at[pl.ds((img * 4 + 3) * NP + base, CHUNK)], l3, sem),
        pltpu.make_async_copy(score.at[pl.ds((img * 2 + 0) * NP + base, CHUNK)], sc0, sem),
        pltpu.make_async_copy(score.at[pl.ds((img * 2 + 1) * NP + base, CHUNK)], sc1, sem),
        pltpu.make_async_copy(bbox.at[pl.ds((img * 4 + 0) * N_GT, N_GT)], bx1, sem),
        pltpu.make_async_copy(bbox.at[pl.ds((img * 4 + 1) * N_GT, N_GT)], by1, sem),
        pltpu.make_async_copy(bbox.at[pl.ds((img * 4 + 2) * N_GT, N_GT)], bx2, sem),
        pltpu.make_async_copy(bbox.at[pl.ds((img * 4 + 3) * N_GT, N_GT)], by2, sem),
    ]
    for cp in copies:
        cp.start()

    def _init_scat(j, _):
        scat[pl.ds(j * L, L)] = jnp.full((L,), -1, _i32)
        return 0

    lax.fori_loop(0, N_GT, _init_scat, 0)

    def _init_flags(i, _):
        flags[pl.ds(i * L, L)] = jnp.zeros((L,), _i32)
        return 0

    lax.fori_loop(0, N_VEC, _init_flags, 0)
    for cp in copies:
        cp.wait()

    def _init_pre(i, _):
        off = i * L
        a1 = ax1[pl.ds(off, L)]
        a2 = ay1[pl.ds(off, L)]
        a3 = ax2[pl.ds(off, L)]
        a4 = ay2[pl.ds(off, L)]
        areaa[pl.ds(off, L)] = (a3 - a1) * (a4 - a2)
        aidxs[pl.ds(off, L)] = base + off + lanes
        return 0

    lax.fori_loop(0, N_VEC, _init_pre, 0)

    for gtb in range(N_GTB):
        blk = (gtb * GTB) // L
        off16 = blk * L
        sub = (gtb * GTB) % L
        b1v = bx1[pl.ds(off16, L)]
        b2v = by1[pl.ds(off16, L)]
        b3v = bx2[pl.ds(off16, L)]
        b4v = by2[pl.ds(off16, L)]
        abv = (b3v - b1v) * (b4v - b2v)
        zsplat = jnp.zeros((L,), _f32)
        gb1 = [zsplat + b1v[sub + j] for j in range(GTB)]
        gb2 = [zsplat + b2v[sub + j] for j in range(GTB)]
        gb3 = [zsplat + b3v[sub + j] for j in range(GTB)]
        gb4 = [zsplat + b4v[sub + j] for j in range(GTB)]
        gab = [zsplat + abv[sub + j] for j in range(GTB)]

        def _main(i, col, gtb=gtb, gb1=gb1, gb2=gb2, gb3=gb3, gb4=gb4,
                  gab=gab):
            off = i * L
            a1 = ax1[pl.ds(off, L)]
            a2 = ay1[pl.ds(off, L)]
            a3 = ax2[pl.ds(off, L)]
            a4 = ay2[pl.ds(off, L)]
            aidx = aidxs[pl.ds(off, L)]
            area_a = areaa[pl.ds(off, L)]
            if gtb == 0:
                rmax = jnp.full((L,), -1.0, _f32)
                rg = jnp.zeros((L,), _i32)
            else:
                rmax = miou[pl.ds(off, L)]
                rg = rga[pl.ds(off, L)]
            cvs = list(col)
            for j in range(GTB):
                g = gtb * GTB + j
                iw = jnp.maximum(
                    jnp.minimum(a3, gb3[j]) - jnp.maximum(a1, gb1[j]), 0.0)
                ih = jnp.maximum(
                    jnp.minimum(a4, gb4[j]) - jnp.maximum(a2, gb2[j]), 0.0)
                inter = iw * ih
                iou = inter / (area_a + gab[j] - inter + 1e-9)
                better = iou > rmax
                rmax = jnp.where(better, iou, rmax)
                rg = jnp.where(better, g, rg)
                cb = iou > cvs[2 * j]
                cvs[2 * j] = jnp.where(cb, iou, cvs[2 * j])
                cvs[2 * j + 1] = jnp.where(cb, aidx, cvs[2 * j + 1])
            miou[pl.ds(off, L)] = rmax
            rga[pl.ds(off, L)] = rg
            return tuple(cvs)

        col0 = []
        for j in range(GTB):
            col0.append(jnp.full((L,), -1.0, _f32))
            col0.append(jnp.zeros((L,), _i32))
        colf = lax.fori_loop(0, N_VEC, _main, tuple(col0))
        for j in range(GTB):
            g = gtb * GTB + j
            colv[pl.ds(g * L, L)] = colf[2 * j]
            coli[pl.ds(g * L, L)] = colf[2 * j + 1]

    def _scatp(i, _):
        off = i * L
        rg = rga[pl.ds(off, L)]
        valid = aidxs[pl.ds(off, L)] < N_ANCHOR
        plsc.store_scatter(scat, [rg * L + lanes],
                           aidxs[pl.ds(off, L)], mask=valid)
        return 0

    lax.fori_loop(0, N_VEC, _scatp, 0)

    TBL = N_GT * L
    pltpu.sync_copy(colv, sh_colv.at[pl.ds(s * TBL, TBL)])
    pltpu.sync_copy(coli, sh_coli.at[pl.ds(s * TBL, TBL)])
    pltpu.sync_copy(scat, sh_scat.at[pl.ds(s * TBL, TBL)])
    plsc.subcore_barrier()

    pltpu.sync_copy(sh_colv, mv)
    pltpu.sync_copy(sh_coli, mi)
    pltpu.sync_copy(sh_scat, ms)

    lane0 = lanes == 0
    zi = jnp.zeros((L,), _i32)

    def _merge(g, _):
        off = g * L
        bv = mv[pl.ds(off, L)]
        bi = mi[pl.ds(off, L)]
        sm = ms[pl.ds(off, L)]
        for ch in range(1, N_CHUNKS):
            coff = ch * TBL + off
            cv = mv[pl.ds(coff, L)]
            ci = mi[pl.ds(coff, L)]
            cb = cv > bv
            bv = jnp.where(cb, cv, bv)
            bi = jnp.where(cb, ci, bi)
            sm = jnp.maximum(sm, ms[pl.ds(coff, L)])
        cmax = jnp.max(bv)
        cidx = jnp.min(jnp.where(bv == cmax, bi, BIG_I))
        sg = jnp.max(sm)
        ga = jnp.where(sg >= 0, sg, cidx)
        plsc.store_scatter(gt_arg, [zi + g], zi + ga, mask=lane0)
        return 0

    lax.fori_loop(0, N_GT, _merge, 0)

    ones_i = jnp.ones((L,), _i32)
    for gb in range(N_GT // L):
        ga_v = gt_arg[pl.ds(gb * L, L)]
        inm = (ga_v >= base) & (ga_v < base + CHUNK)
        li = jnp.where(inm, ga_v - base, 0)
        plsc.store_scatter(flags, [li], ones_i, mask=inm)

    def _delta(i, acc):
        a_pos, a_rl, a_val, a_ce = acc
        off = i * L
        valid = aidxs[pl.ds(off, L)] < N_ANCHOR
        fl = flags[pl.ds(off, L)] > 0
        mi_v = miou[pl.ds(off, L)]
        posm = (mi_v >= POS_IOU) | fl
        validm = posm | ((mi_v < NEG_IOU) & valid)
        rg = rga[pl.ds(off, L)]
        a1 = ax1[pl.ds(off, L)]
        a2 = ay1[pl.ds(off, L)]
        a3 = ax2[pl.ds(off, L)]
        a4 = ay2[pl.ds(off, L)]
        m1 = plsc.load_gather(bx1, [rg])
        m2 = plsc.load_gather(by1, [rg])
        m3 = plsc.load_gather(bx2, [rg])
        m4 = plsc.load_gather(by2, [rg])
        eps = jnp.finfo(_f32).eps
        w = a3 - a1
        h = a4 - a2
        cx = a1 + w * 0.5
        cy = a2 + h * 0.5
        dw_ = m3 - m1
        dh_ = m4 - m2
        dcx = m1 + dw_ * 0.5
        dcy = m2 + dh_ * 0.5
        w = jnp.maximum(w, eps)
        h = jnp.maximum(h, eps)
        tdx = (dcx - cx) / w
        tdy = (dcy - cy) / h
        tdw = _vlog(dw_ / w)
        tdh = _vlog(dh_ / h)
        rl = (_sl1(jnp.abs(tdx - l0[pl.ds(off, L)]))
              + _sl1(jnp.abs(tdy - l1[pl.ds(off, L)]))
              + _sl1(jnp.abs(tdw - l2[pl.ds(off, L)]))
              + _sl1(jnp.abs(tdh - l3[pl.ds(off, L)])))
        s0 = sc0[pl.ds(off, L)]
        s1 = sc1[pl.ds(off, L)]
        mx = jnp.maximum(s0, s1)
        lse = mx + _vlog(1.0 + jnp.exp(-jnp.abs(s0 - s1)))
        a_pos = a_pos + jnp.where(posm, 1.0, 0.0)
        a_rl = a_rl + jnp.where(posm, rl, 0.0)
        a_val = a_val + jnp.where(validm, 1.0, 0.0)
        ce = lse - jnp.where(posm, s1, s0)
        a_ce = a_ce + jnp.where(validm, ce, 0.0)
        return (a_pos, a_rl, a_val, a_ce)

    zero = jnp.zeros((L,), _f32)
    a_pos, a_rl, a_val, a_ce = lax.fori_loop(
        0, N_VEC, _delta, (zero, zero, zero, zero))
    stage[pl.ds(0, L)] = a_pos
    stage[pl.ds(L, L)] = a_rl
    stage[pl.ds(2 * L, L)] = a_val
    stage[pl.ds(3 * L, L)] = a_ce
    pltpu.sync_copy(stage, sh_sums.at[pl.ds(s * 4 * L, 4 * L)])
    plsc.subcore_barrier()

    @pl.when(s == 0)
    def _finalize():
        pltpu.sync_copy(sh_sums, fin)
        t_pos = jnp.zeros((L,), _f32)
        t_rl = jnp.zeros((L,), _f32)
        t_val = jnp.zeros((L,), _f32)
        t_ce = jnp.zeros((L,), _f32)
        for ch in range(N_CHUNKS):
            o = ch * 4 * L
            t_pos = t_pos + fin[pl.ds(o, L)]
            t_rl = t_rl + fin[pl.ds(o + L, L)]
            t_val = t_val + fin[pl.ds(o + 2 * L, L)]
            t_ce = t_ce + fin[pl.ds(o + 3 * L, L)]
        zf = jnp.zeros((L,), _f32)
        num_pos = jnp.maximum(zf + jnp.sum(t_pos), 1.0)
        num_val = jnp.maximum(zf + jnp.sum(t_val), 1.0)
        total = ((zf + jnp.sum(t_rl)) / num_pos
                 + (zf + jnp.sum(t_ce)) / num_val)
        outv[...] = total
        pltpu.sync_copy(outv, out.at[pl.ds(c * L, L)])


def _sc_call(anc, bbox, loc, score):
    mesh = plsc.VectorSubcoreMesh(core_axis_name="c", subcore_axis_name="s",
                                  num_cores=2, num_subcores=16)
    return pl.kernel(
        _sc_body,
        out_type=jax.ShapeDtypeStruct((2 * L,), _f32),
        mesh=mesh,
        compiler_params=pltpu.CompilerParams(needs_layout_passes=False),
        scratch_types=[
            pltpu.VMEM((CHUNK,), _f32), pltpu.VMEM((CHUNK,), _f32),
            pltpu.VMEM((CHUNK,), _f32), pltpu.VMEM((CHUNK,), _f32),
            pltpu.VMEM((CHUNK,), _f32), pltpu.VMEM((CHUNK,), _f32),
            pltpu.VMEM((CHUNK,), _f32), pltpu.VMEM((CHUNK,), _f32),
            pltpu.VMEM((CHUNK,), _f32), pltpu.VMEM((CHUNK,), _f32),
            pltpu.VMEM((N_GT,), _f32), pltpu.VMEM((N_GT,), _f32),
            pltpu.VMEM((N_GT,), _f32), pltpu.VMEM((N_GT,), _f32),
            pltpu.VMEM((CHUNK,), _f32), pltpu.VMEM((CHUNK,), _i32),
            pltpu.VMEM((CHUNK,), _f32), pltpu.VMEM((CHUNK,), _i32),
            pltpu.VMEM((CHUNK,), _i32),
            pltpu.VMEM((N_GT * L,), _f32), pltpu.VMEM((N_GT * L,), _i32),
            pltpu.VMEM((N_GT * L,), _i32),
            pltpu.VMEM((N_CHUNKS * N_GT * L,), _f32),
            pltpu.VMEM((N_CHUNKS * N_GT * L,), _i32),
            pltpu.VMEM((N_CHUNKS * N_GT * L,), _i32),
            pltpu.VMEM((N_GT,), _i32),
            pltpu.VMEM((4 * L,), _f32),
            pltpu.VMEM((N_CHUNKS * 4 * L,), _f32),
            pltpu.VMEM((L,), _f32),
            pltpu.SemaphoreType.DMA,
            pltpu.VMEM_SHARED((N_CHUNKS * N_GT * L,), _f32),
            pltpu.VMEM_SHARED((N_CHUNKS * N_GT * L,), _i32),
            pltpu.VMEM_SHARED((N_CHUNKS * N_GT * L,), _i32),
            pltpu.VMEM_SHARED((N_CHUNKS * 4 * L,), _f32),
        ],
    )(anc, bbox, loc, score)


def _tc_body(anchors_ref, bbox_ref, loc_ref, score_ref, out_ref):
    NP = N_PAD_TC
    ax1 = anchors_ref[0:1, :]
    ay1 = anchors_ref[1:2, :]
    ax2 = anchors_ref[2:3, :]
    ay2 = anchors_ref[3:4, :]
    bt = bbox_ref[0]
    bx1 = bt[0:1, :].reshape(N_GT, 1)
    by1 = bt[1:2, :].reshape(N_GT, 1)
    bx2 = bt[2:3, :].reshape(N_GT, 1)
    by2 = bt[3:4, :].reshape(N_GT, 1)

    tlx = jnp.maximum(ax1, bx1)
    tly = jnp.maximum(ay1, by1)
    brx = jnp.minimum(ax2, bx2)
    bry = jnp.minimum(ay2, by2)
    iw = jnp.maximum(brx - tlx, 0.0)
    ih = jnp.maximum(bry - tly, 0.0)
    inter = iw * ih
    area_a = (ax2 - ax1) * (ay2 - ay1)
    area_b = (bx2 - bx1) * (by2 - by1)
    iou = inter / (area_a + area_b - inter + 1e-9)

    i_iota = lax.broadcasted_iota(jnp.int32, (N_GT, NP), 1)
    g_iota = lax.broadcasted_iota(jnp.int32, (N_GT, NP), 0)

    max_iou = jnp.max(iou, axis=0, keepdims=True)
    argmax_g = jnp.min(jnp.where(iou == max_iou, g_iota, N_GT),
                       axis=0, keepdims=True)

    colmax = jnp.max(iou, axis=1, keepdims=True)
    col_argmax = jnp.min(jnp.where(iou == colmax, i_iota, NP),
                         axis=1, keepdims=True)

    lane_valid = i_iota < N_ANCHOR
    scat = jnp.max(jnp.where((argmax_g == g_iota) & lane_valid, i_iota, -1),
                   axis=1, keepdims=True)
    gt_argmax = jnp.where(scat >= 0, scat, col_argmax)

    valid_lane = lax.broadcasted_iota(jnp.int32, (1, NP), 1) < N_ANCHOR
    member = jnp.max(jnp.where(gt_argmax == i_iota, 1, 0),
                     axis=0, keepdims=True) > 0
    pos = (max_iou >= POS_IOU) | member
    neg = (max_iou < NEG_IOU) & valid_lane
    valid = pos | neg

    onehot = argmax_g == g_iota
    mx1 = jnp.max(jnp.where(onehot, bx1, -1e30), axis=0, keepdims=True)
    my1 = jnp.max(jnp.where(onehot, by1, -1e30), axis=0, keepdims=True)
    mx2 = jnp.max(jnp.where(onehot, bx2, -1e30), axis=0, keepdims=True)
    my2 = jnp.max(jnp.where(onehot, by2, -1e30), axis=0, keepdims=True)

    eps = jnp.finfo(jnp.float32).eps
    w = ax2 - ax1
    h = ay2 - ay1
    cx = ax1 + w * 0.5
    cy = ay1 + h * 0.5
    dw_ = mx2 - mx1
    dh_ = my2 - my1
    dcx = mx1 + dw_ * 0.5
    dcy = my1 + dh_ * 0.5
    w = jnp.maximum(w, eps)
    h = jnp.maximum(h, eps)
    tdx = (dcx - cx) / w
    tdy = (dcy - cy) / h
    tdw = jnp.log(dw_ / w)
    tdh = jnp.log(dh_ / h)

    lr = loc_ref[0]
    d0 = jnp.abs(tdx - lr[0:1, :])
    d1 = jnp.abs(tdy - lr[1:2, :])
    d2 = jnp.abs(tdw - lr[2:3, :])
    d3 = jnp.abs(tdh - lr[3:4, :])
    rl = _sl1(d0) + _sl1(d1) + _sl1(d2) + _sl1(d3)
    posf = pos.astype(jnp.float32)
    num_pos = jnp.maximum(jnp.sum(posf), 1.0)
    loc_loss = jnp.sum(rl * posf) / num_pos

    s0 = score_ref[0][0:1, :]
    s1 = score_ref[0][1:2, :]
    m = jnp.maximum(s0, s1)
    lse = m + jnp.log(jnp.exp(s0 - m) + jnp.exp(s1 - m))
    ce = lse - jnp.where(pos, s1, s0)
    validf = valid.astype(jnp.float32)
    num_valid = jnp.maximum(jnp.sum(validf), 1.0)
    cls_loss = jnp.sum(jnp.where(valid, ce, 0.0)) / num_valid

    out_ref[:, :, :] = (loc_loss + cls_loss).reshape(1, 1, 1)


def _tc_call(anchors_t, bboxes_t, loc_t, score_t, n_img):
    return pl.pallas_call(
        _tc_body,
        grid=(n_img,),
        in_specs=[
            pl.BlockSpec((4, N_PAD_TC), lambda b: (0, 0)),
            pl.BlockSpec((1, 4, N_GT), lambda b: (b, 0, 0)),
            pl.BlockSpec((1, 4, N_PAD_TC), lambda b: (b, 0, 0)),
            pl.BlockSpec((1, 2, N_PAD_TC), lambda b: (b, 0, 0)),
        ],
        out_specs=pl.BlockSpec((1, 1, 1), lambda b: (b, 0, 0)),
        out_shape=jax.ShapeDtypeStruct((n_img, 1, 1), _f32),
    )(anchors_t, bboxes_t, loc_t, score_t)


@jax.jit
def kernel(anchors, bboxes, rpn_loc, rpn_score):
    anchors = anchors.astype(_f32)
    bboxes = bboxes.astype(_f32)
    pad_sc = N_PAD_SC - N_ANCHOR
    anc_sc = jnp.pad(anchors.T, ((0, 0), (0, pad_sc))).reshape(-1)
    bbox_sc = jnp.transpose(bboxes[2:], (0, 2, 1)).reshape(-1)
    loc_sc = jnp.pad(jnp.transpose(rpn_loc[2:], (0, 2, 1)),
                     ((0, 0), (0, 0), (0, pad_sc))).reshape(-1)
    score_sc = jnp.pad(jnp.transpose(rpn_score[2:], (0, 2, 1)),
                       ((0, 0), (0, 0), (0, pad_sc))).reshape(-1)
    sc_out = _sc_call(anc_sc, bbox_sc, loc_sc, score_sc)

    pad_tc = N_PAD_TC - N_ANCHOR
    anc_tc = jnp.pad(anchors.T, ((0, 0), (0, pad_tc)))
    bbox_tc = jnp.transpose(bboxes[:2], (0, 2, 1))
    loc_tc = jnp.pad(jnp.transpose(rpn_loc[:2], (0, 2, 1)),
                     ((0, 0), (0, 0), (0, pad_tc)))
    score_tc = jnp.pad(jnp.transpose(rpn_score[:2], (0, 2, 1)),
                       ((0, 0), (0, 0), (0, pad_tc)))
    tc_out = _tc_call(anc_tc, bbox_tc, loc_tc, score_tc, 2)

    return tc_out[0, 0, 0] + tc_out[1, 0, 0] + sc_out[0] + sc_out[L]


# speedup vs baseline: 2.5610x; 1.0020x over previous
import functools

import jax
import jax.numpy as jnp
from jax import lax
from jax.experimental import pallas as pl
from jax.experimental.pallas import tpu as pltpu
from jax.experimental.pallas import tpu_sc as plsc

N_ANCHOR = 20000
N_GT = 64
POS_IOU = 0.7
NEG_IOU = 0.3
BIG_I = 2**30

L = 16
N_CHUNKS = 16
CHUNK = 1264
N_VEC = CHUNK // L
GTB = 4
N_GTB = N_GT // GTB
N_PAD_SC = N_CHUNKS * CHUNK

N_PAD_TC = 20480

_f32 = jnp.float32
_i32 = jnp.int32
_LN2 = 0.6931471805599453
_SQRT2 = 1.4142135623730951


def _vlog(x):
    bits = plsc.bitcast(x, _i32)
    e = jnp.right_shift(bits, 23) - 127
    m = plsc.bitcast(jnp.bitwise_or(jnp.bitwise_and(bits, 0x7FFFFF),
                                    0x3F800000), _f32)
    big = m > _SQRT2
    m = jnp.where(big, m * 0.5, m)
    e = jnp.where(big, e + 1, e)
    z = (m - 1.0) / (m + 1.0)
    z2 = z * z
    p = ((z2 * (1.0 / 7.0) + (1.0 / 5.0)) * z2 + (1.0 / 3.0)) * z2 + 1.0
    return e.astype(_f32) * _LN2 + 2.0 * z * p


def _sl1(d):
    return jnp.where(d < 1.0, 0.5 * d * d, d - 0.5)


def _sc_body(anc, bbox, loc, score, out,
             ax1, ay1, ax2, ay2, l0, l1, l2, l3, sc0, sc1,
             bx1, by1, bx2, by2, areaa, aidxs,
             miou, rga, flags, colv, coli, scat,
             mv, mi, ms, gt_arg, stage, fin, outv, sem,
             sh_colv, sh_coli, sh_scat, sh_sums):
    c = lax.axis_index("c")
    s = lax.axis_index("s")
    img = c
    base = s * CHUNK
    lanes = lax.broadcasted_iota(_i32, (L,), 0)

    NP = N_PAD_SC
    copies = [
        pltpu.make_async_copy(anc.at[pl.ds(0 * NP + base, CHUNK)], ax1, sem),
        pltpu.make_async_copy(anc.at[pl.ds(1 * NP + base, CHUNK)], ay1, sem),
        pltpu.make_async_copy(anc.at[pl.ds(2 * NP + base, CHUNK)], ax2, sem),
        pltpu.make_async_copy(anc.at[pl.ds(3 * NP + base, CHUNK)], ay2, sem),
        pltpu.make_async_copy(loc.at[pl.ds((img * 4 + 0) * NP + base, CHUNK)], l0, sem),
        pltpu.make_async_copy(loc.at[pl.ds((img * 4 + 1) * NP + base, CHUNK)], l1, sem),
        pltpu.make_async_copy(loc.at[pl.ds((img * 4 + 2) * NP + base, CHUNK)], l2, sem),
        pltpu.make_async_copy(loc.at[pl.ds((img * 4 + 3) * NP + base, CHUNK)], l3, sem),
        pltpu.make_async_copy(score.at[pl.ds((img * 2 + 0) * NP + base, CHUNK)], sc0, sem),
        pltpu.make_async_copy(score.at[pl.ds((img * 2 + 1) * NP + base, CHUNK)], sc1, sem),
        pltpu.make_async_copy(bbox.at[pl.ds((img * 4 + 0) * N_GT, N_GT)], bx1, sem),
        pltpu.make_async_copy(bbox.at[pl.ds((img * 4 + 1) * N_GT, N_GT)], by1, sem),
        pltpu.make_async_copy(bbox.at[pl.ds((img * 4 + 2) * N_GT, N_GT)], bx2, sem),
        pltpu.make_async_copy(bbox.at[pl.ds((img * 4 + 3) * N_GT, N_GT)], by2, sem),
    ]
    for cp in copies:
        cp.start()

    def _init_scat(j, _):
        scat[pl.ds(j * L, L)] = jnp.full((L,), -1, _i32)
        return 0

    lax.fori_loop(0, N_GT, _init_scat, 0)

    def _init_flags(i, _):
        flags[pl.ds(i * L, L)] = jnp.zeros((L,), _i32)
        return 0

    lax.fori_loop(0, N_VEC, _init_flags, 0)
    for cp in copies:
        cp.wait()

    def _init_pre(i, _):
        off = i * L
        a1 = ax1[pl.ds(off, L)]
        a2 = ay1[pl.ds(off, L)]
        a3 = ax2[pl.ds(off, L)]
        a4 = ay2[pl.ds(off, L)]
        areaa[pl.ds(off, L)] = (a3 - a1) * (a4 - a2)
        aidxs[pl.ds(off, L)] = base + off + lanes
        return 0

    lax.fori_loop(0, N_VEC, _init_pre, 0)

    for gtb in range(N_GTB):
        blk = (gtb * GTB) // L
        off16 = blk * L
        sub = (gtb * GTB) % L
        b1v = bx1[pl.ds(off16, L)]
        b2v = by1[pl.ds(off16, L)]
        b3v = bx2[pl.ds(off16, L)]
        b4v = by2[pl.ds(off16, L)]
        abv = (b3v - b1v) * (b4v - b2v)
        zsplat = jnp.zeros((L,), _f32)
        gb1 = [zsplat + b1v[sub + j] for j in range(GTB)]
        gb2 = [zsplat + b2v[sub + j] for j in range(GTB)]
        gb3 = [zsplat + b3v[sub + j] for j in range(GTB)]
        gb4 = [zsplat + b4v[sub + j] for j in range(GTB)]
        gab = [zsplat + abv[sub + j] for j in range(GTB)]

        def _main(i, col, gtb=gtb, gb1=gb1, gb2=gb2, gb3=gb3, gb4=gb4,
                  gab=gab):
            off = i * L
            a1 = ax1[pl.ds(off, L)]
            a2 = ay1[pl.ds(off, L)]
            a3 = ax2[pl.ds(off, L)]
            a4 = ay2[pl.ds(off, L)]
            aidx = aidxs[pl.ds(off, L)]
            area_a = areaa[pl.ds(off, L)]
            if gtb == 0:
                rmax = jnp.full((L,), -1.0, _f32)
                rg = jnp.zeros((L,), _i32)
            else:
                rmax = miou[pl.ds(off, L)]
                rg = rga[pl.ds(off, L)]
            cvs = list(col)
            for j in range(GTB):
                g = gtb * GTB + j
                iw = jnp.maximum(
                    jnp.minimum(a3, gb3[j]) - jnp.maximum(a1, gb1[j]), 0.0)
                ih = jnp.maximum(
                    jnp.minimum(a4, gb4[j]) - jnp.maximum(a2, gb2[j]), 0.0)
                inter = iw * ih
                iou = inter / (area_a + gab[j] - inter + 1e-9)
                better = iou > rmax
                rmax = jnp.where(better, iou, rmax)
                rg = jnp.where(better, g, rg)
                cb = iou > cvs[2 * j]
                cvs[2 * j] = jnp.where(cb, iou, cvs[2 * j])
                cvs[2 * j + 1] = jnp.where(cb, aidx, cvs[2 * j + 1])
            miou[pl.ds(off, L)] = rmax
            rga[pl.ds(off, L)] = rg
            return tuple(cvs)

        col0 = []
        for j in range(GTB):
            col0.append(jnp.full((L,), -1.0, _f32))
            col0.append(jnp.zeros((L,), _i32))
        colf = lax.fori_loop(0, N_VEC, _main, tuple(col0))
        for j in range(GTB):
            g = gtb * GTB + j
            colv[pl.ds(g * L, L)] = colf[2 * j]
            coli[pl.ds(g * L, L)] = colf[2 * j + 1]

    def _scatp(i, _):
        off = i * L
        rg = rga[pl.ds(off, L)]
        valid = aidxs[pl.ds(off, L)] < N_ANCHOR
        plsc.store_scatter(scat, [rg * L + lanes],
                           aidxs[pl.ds(off, L)], mask=valid)
        return 0

    lax.fori_loop(0, N_VEC, _scatp, 0)

    TBL = N_GT * L
    pltpu.sync_copy(colv, sh_colv.at[pl.ds(s * TBL, TBL)])
    pltpu.sync_copy(coli, sh_coli.at[pl.ds(s * TBL, TBL)])
    pltpu.sync_copy(scat, sh_scat.at[pl.ds(s * TBL, TBL)])
    plsc.subcore_barrier()

    pltpu.sync_copy(sh_colv, mv)
    pltpu.sync_copy(sh_coli, mi)
    pltpu.sync_copy(sh_scat, ms)

    lane0 = lanes == 0
    zi = jnp.zeros((L,), _i32)

    def _merge(g, _):
        off = g * L
        sm = ms[pl.ds(off, L)]
        for ch in range(1, N_CHUNKS):
            sm = jnp.maximum(sm, ms[pl.ds(ch * TBL + off, L)])
        sg = jnp.max(sm)
        plsc.store_scatter(gt_arg, [zi + g], zi + sg, mask=lane0 & (sg >= 0))

        @pl.when(sg < 0)
        def _col():
            bv = mv[pl.ds(off, L)]
            bi = mi[pl.ds(off, L)]
            for ch in range(1, N_CHUNKS):
                coff = ch * TBL + off
                cv = mv[pl.ds(coff, L)]
                ci = mi[pl.ds(coff, L)]
                cb = cv > bv
                bv = jnp.where(cb, cv, bv)
                bi = jnp.where(cb, ci, bi)
            cmax = jnp.max(bv)
            cidx = jnp.min(jnp.where(bv == cmax, bi, BIG_I))
            plsc.store_scatter(gt_arg, [zi + g], zi + cidx, mask=lane0)

        return 0

    lax.fori_loop(0, N_GT, _merge, 0)

    ones_i = jnp.ones((L,), _i32)
    for gb in range(N_GT // L):
        ga_v = gt_arg[pl.ds(gb * L, L)]
        inm = (ga_v >= base) & (ga_v < base + CHUNK)
        li = jnp.where(inm, ga_v - base, 0)
        plsc.store_scatter(flags, [li], ones_i, mask=inm)

    def _delta(i, acc):
        a_pos, a_rl, a_val, a_ce = acc
        off = i * L
        valid = aidxs[pl.ds(off, L)] < N_ANCHOR
        fl = flags[pl.ds(off, L)] > 0
        mi_v = miou[pl.ds(off, L)]
        posm = (mi_v >= POS_IOU) | fl
        validm = posm | ((mi_v < NEG_IOU) & valid)
        rg = rga[pl.ds(off, L)]
        a1 = ax1[pl.ds(off, L)]
        a2 = ay1[pl.ds(off, L)]
        a3 = ax2[pl.ds(off, L)]
        a4 = ay2[pl.ds(off, L)]
        m1 = plsc.load_gather(bx1, [rg])
        m2 = plsc.load_gather(by1, [rg])
        m3 = plsc.load_gather(bx2, [rg])
        m4 = plsc.load_gather(by2, [rg])
        eps = jnp.finfo(_f32).eps
        w = a3 - a1
        h = a4 - a2
        cx = a1 + w * 0.5
        cy = a2 + h * 0.5
        dw_ = m3 - m1
        dh_ = m4 - m2
        dcx = m1 + dw_ * 0.5
        dcy = m2 + dh_ * 0.5
        w = jnp.maximum(w, eps)
        h = jnp.maximum(h, eps)
        tdx = (dcx - cx) / w
        tdy = (dcy - cy) / h
        tdw = _vlog(dw_ / w)
        tdh = _vlog(dh_ / h)
        rl = (_sl1(jnp.abs(tdx - l0[pl.ds(off, L)]))
              + _sl1(jnp.abs(tdy - l1[pl.ds(off, L)]))
              + _sl1(jnp.abs(tdw - l2[pl.ds(off, L)]))
              + _sl1(jnp.abs(tdh - l3[pl.ds(off, L)])))
        s0 = sc0[pl.ds(off, L)]
        s1 = sc1[pl.ds(off, L)]
        mx = jnp.maximum(s0, s1)
        lse = mx + _vlog(1.0 + jnp.exp(-jnp.abs(s0 - s1)))
        a_pos = a_pos + jnp.where(posm, 1.0, 0.0)
        a_rl = a_rl + jnp.where(posm, rl, 0.0)
        a_val = a_val + jnp.where(validm, 1.0, 0.0)
        ce = lse - jnp.where(posm, s1, s0)
        a_ce = a_ce + jnp.where(validm, ce, 0.0)
        return (a_pos, a_rl, a_val, a_ce)

    zero = jnp.zeros((L,), _f32)
    a_pos, a_rl, a_val, a_ce = lax.fori_loop(
        0, N_VEC, _delta, (zero, zero, zero, zero))
    stage[pl.ds(0, L)] = a_pos
    stage[pl.ds(L, L)] = a_rl
    stage[pl.ds(2 * L, L)] = a_val
    stage[pl.ds(3 * L, L)] = a_ce
    pltpu.sync_copy(stage, sh_sums.at[pl.ds(s * 4 * L, 4 * L)])
    plsc.subcore_barrier()

    @pl.when(s == 0)
    def _finalize():
        pltpu.sync_copy(sh_sums, fin)
        t_pos = jnp.zeros((L,), _f32)
        t_rl = jnp.zeros((L,), _f32)
        t_val = jnp.zeros((L,), _f32)
        t_ce = jnp.zeros((L,), _f32)
        for ch in range(N_CHUNKS):
            o = ch * 4 * L
            t_pos = t_pos + fin[pl.ds(o, L)]
            t_rl = t_rl + fin[pl.ds(o + L, L)]
            t_val = t_val + fin[pl.ds(o + 2 * L, L)]
            t_ce = t_ce + fin[pl.ds(o + 3 * L, L)]
        zf = jnp.zeros((L,), _f32)
        num_pos = jnp.maximum(zf + jnp.sum(t_pos), 1.0)
        num_val = jnp.maximum(zf + jnp.sum(t_val), 1.0)
        total = ((zf + jnp.sum(t_rl)) / num_pos
                 + (zf + jnp.sum(t_ce)) / num_val)
        outv[...] = total
        pltpu.sync_copy(outv, out.at[pl.ds(c * L, L)])


def _sc_call(anc, bbox, loc, score):
    mesh = plsc.VectorSubcoreMesh(core_axis_name="c", subcore_axis_name="s",
                                  num_cores=2, num_subcores=16)
    return pl.kernel(
        _sc_body,
        out_type=jax.ShapeDtypeStruct((2 * L,), _f32),
        mesh=mesh,
        compiler_params=pltpu.CompilerParams(needs_layout_passes=False),
        scratch_types=[
            pltpu.VMEM((CHUNK,), _f32), pltpu.VMEM((CHUNK,), _f32),
            pltpu.VMEM((CHUNK,), _f32), pltpu.VMEM((CHUNK,), _f32),
            pltpu.VMEM((CHUNK,), _f32), pltpu.VMEM((CHUNK,), _f32),
            pltpu.VMEM((CHUNK,), _f32), pltpu.VMEM((CHUNK,), _f32),
            pltpu.VMEM((CHUNK,), _f32), pltpu.VMEM((CHUNK,), _f32),
            pltpu.VMEM((N_GT,), _f32), pltpu.VMEM((N_GT,), _f32),
            pltpu.VMEM((N_GT,), _f32), pltpu.VMEM((N_GT,), _f32),
            pltpu.VMEM((CHUNK,), _f32), pltpu.VMEM((CHUNK,), _i32),
            pltpu.VMEM((CHUNK,), _f32), pltpu.VMEM((CHUNK,), _i32),
            pltpu.VMEM((CHUNK,), _i32),
            pltpu.VMEM((N_GT * L,), _f32), pltpu.VMEM((N_GT * L,), _i32),
            pltpu.VMEM((N_GT * L,), _i32),
            pltpu.VMEM((N_CHUNKS * N_GT * L,), _f32),
            pltpu.VMEM((N_CHUNKS * N_GT * L,), _i32),
            pltpu.VMEM((N_CHUNKS * N_GT * L,), _i32),
            pltpu.VMEM((N_GT,), _i32),
            pltpu.VMEM((4 * L,), _f32),
            pltpu.VMEM((N_CHUNKS * 4 * L,), _f32),
            pltpu.VMEM((L,), _f32),
            pltpu.SemaphoreType.DMA,
            pltpu.VMEM_SHARED((N_CHUNKS * N_GT * L,), _f32),
            pltpu.VMEM_SHARED((N_CHUNKS * N_GT * L,), _i32),
            pltpu.VMEM_SHARED((N_CHUNKS * N_GT * L,), _i32),
            pltpu.VMEM_SHARED((N_CHUNKS * 4 * L,), _f32),
        ],
    )(anc, bbox, loc, score)


def _tc_body(anchors_ref, bbox_ref, loc_ref, score_ref, out_ref):
    NP = N_PAD_TC
    ax1 = anchors_ref[0:1, :]
    ay1 = anchors_ref[1:2, :]
    ax2 = anchors_ref[2:3, :]
    ay2 = anchors_ref[3:4, :]
    bt = bbox_ref[0]
    bx1 = bt[0:1, :].reshape(N_GT, 1)
    by1 = bt[1:2, :].reshape(N_GT, 1)
    bx2 = bt[2:3, :].reshape(N_GT, 1)
    by2 = bt[3:4, :].reshape(N_GT, 1)

    tlx = jnp.maximum(ax1, bx1)
    tly = jnp.maximum(ay1, by1)
    brx = jnp.minimum(ax2, bx2)
    bry = jnp.minimum(ay2, by2)
    iw = jnp.maximum(brx - tlx, 0.0)
    ih = jnp.maximum(bry - tly, 0.0)
    inter = iw * ih
    area_a = (ax2 - ax1) * (ay2 - ay1)
    area_b = (bx2 - bx1) * (by2 - by1)
    iou = inter / (area_a + area_b - inter + 1e-9)

    i_iota = lax.broadcasted_iota(jnp.int32, (N_GT, NP), 1)
    g_iota = lax.broadcasted_iota(jnp.int32, (N_GT, NP), 0)

    max_iou = jnp.max(iou, axis=0, keepdims=True)
    argmax_g = jnp.min(jnp.where(iou == max_iou, g_iota, N_GT),
                       axis=0, keepdims=True)

    colmax = jnp.max(iou, axis=1, keepdims=True)
    col_argmax = jnp.min(jnp.where(iou == colmax, i_iota, NP),
                         axis=1, keepdims=True)

    lane_valid = i_iota < N_ANCHOR
    scat = jnp.max(jnp.where((argmax_g == g_iota) & lane_valid, i_iota, -1),
                   axis=1, keepdims=True)
    gt_argmax = jnp.where(scat >= 0, scat, col_argmax)

    valid_lane = lax.broadcasted_iota(jnp.int32, (1, NP), 1) < N_ANCHOR
    member = jnp.max(jnp.where(gt_argmax == i_iota, 1, 0),
                     axis=0, keepdims=True) > 0
    pos = (max_iou >= POS_IOU) | member
    neg = (max_iou < NEG_IOU) & valid_lane
    valid = pos | neg

    onehot = argmax_g == g_iota
    mx1 = jnp.max(jnp.where(onehot, bx1, -1e30), axis=0, keepdims=True)
    my1 = jnp.max(jnp.where(onehot, by1, -1e30), axis=0, keepdims=True)
    mx2 = jnp.max(jnp.where(onehot, bx2, -1e30), axis=0, keepdims=True)
    my2 = jnp.max(jnp.where(onehot, by2, -1e30), axis=0, keepdims=True)

    eps = jnp.finfo(jnp.float32).eps
    w = ax2 - ax1
    h = ay2 - ay1
    cx = ax1 + w * 0.5
    cy = ay1 + h * 0.5
    dw_ = mx2 - mx1
    dh_ = my2 - my1
    dcx = mx1 + dw_ * 0.5
    dcy = my1 + dh_ * 0.5
    w = jnp.maximum(w, eps)
    h = jnp.maximum(h, eps)
    tdx = (dcx - cx) / w
    tdy = (dcy - cy) / h
    tdw = jnp.log(dw_ / w)
    tdh = jnp.log(dh_ / h)

    lr = loc_ref[0]
    d0 = jnp.abs(tdx - lr[0:1, :])
    d1 = jnp.abs(tdy - lr[1:2, :])
    d2 = jnp.abs(tdw - lr[2:3, :])
    d3 = jnp.abs(tdh - lr[3:4, :])
    rl = _sl1(d0) + _sl1(d1) + _sl1(d2) + _sl1(d3)
    posf = pos.astype(jnp.float32)
    num_pos = jnp.maximum(jnp.sum(posf), 1.0)
    loc_loss = jnp.sum(rl * posf) / num_pos

    s0 = score_ref[0][0:1, :]
    s1 = score_ref[0][1:2, :]
    m = jnp.maximum(s0, s1)
    lse = m + jnp.log(jnp.exp(s0 - m) + jnp.exp(s1 - m))
    ce = lse - jnp.where(pos, s1, s0)
    validf = valid.astype(jnp.float32)
    num_valid = jnp.maximum(jnp.sum(validf), 1.0)
    cls_loss = jnp.sum(jnp.where(valid, ce, 0.0)) / num_valid

    out_ref[:, :, :] = (loc_loss + cls_loss).reshape(1, 1, 1)


def _tc_call(anchors_t, bboxes_t, loc_t, score_t, n_img):
    return pl.pallas_call(
        _tc_body,
        grid=(n_img,),
        in_specs=[
            pl.BlockSpec((4, N_PAD_TC), lambda b: (0, 0)),
            pl.BlockSpec((1, 4, N_GT), lambda b: (b, 0, 0)),
            pl.BlockSpec((1, 4, N_PAD_TC), lambda b: (b, 0, 0)),
            pl.BlockSpec((1, 2, N_PAD_TC), lambda b: (b, 0, 0)),
        ],
        out_specs=pl.BlockSpec((1, 1, 1), lambda b: (b, 0, 0)),
        out_shape=jax.ShapeDtypeStruct((n_img, 1, 1), _f32),
    )(anchors_t, bboxes_t, loc_t, score_t)


@jax.jit
def kernel(anchors, bboxes, rpn_loc, rpn_score):
    anchors = anchors.astype(_f32)
    bboxes = bboxes.astype(_f32)
    pad_sc = N_PAD_SC - N_ANCHOR
    anc_sc = jnp.pad(anchors.T, ((0, 0), (0, pad_sc))).reshape(-1)
    bbox_sc = jnp.transpose(bboxes[2:], (0, 2, 1)).reshape(-1)
    loc_sc = jnp.pad(jnp.transpose(rpn_loc[2:], (0, 2, 1)),
                     ((0, 0), (0, 0), (0, pad_sc))).reshape(-1)
    score_sc = jnp.pad(jnp.transpose(rpn_score[2:], (0, 2, 1)),
                       ((0, 0), (0, 0), (0, pad_sc))).reshape(-1)
    sc_out = _sc_call(anc_sc, bbox_sc, loc_sc, score_sc)

    pad_tc = N_PAD_TC - N_ANCHOR
    anc_tc = jnp.pad(anchors.T, ((0, 0), (0, pad_tc)))
    bbox_tc = jnp.transpose(bboxes[:2], (0, 2, 1))
    loc_tc = jnp.pad(jnp.transpose(rpn_loc[:2], (0, 2, 1)),
                     ((0, 0), (0, 0), (0, pad_tc)))
    score_tc = jnp.pad(jnp.transpose(rpn_score[:2], (0, 2, 1)),
                       ((0, 0), (0, 0), (0, pad_tc)))
    tc_out = _tc_call(anc_tc, bbox_tc, loc_tc, score_tc, 2)

    return tc_out[0, 0, 0] + tc_out[1, 0, 0] + sc_out[0] + sc_out[L]
